# baseline (speedup 1.0000x reference)
import numpy as np

import jax
import jax.numpy as jnp
from jax.experimental import pallas as pl
from jax.experimental.pallas import tpu as pltpu

N = 10
BB = 256


def _elu(x):
    return jnp.where(x > 0, x, jnp.exp(x) - 1.0)


def _leaky(x):
    return jnp.where(x > 0, x, 0.2 * x)


def _dot(a, b):
    return jax.lax.dot_general(a, b, (((1,), (0,)), ((), ())),
                               preferred_element_type=jnp.float32)


def _ln(x, g, b, om):
    m = _dot(x, om)
    d = x - m
    v = _dot(d * d, om)
    return d * jax.lax.rsqrt(v + 1e-5) * g + b


def _node_scores(wsd, feat3):
    return jax.lax.dot_general(wsd, feat3, (((1,), (2,)), ((), ())),
                               preferred_element_type=jnp.float32)


def _gat_attention(vals3, es, ed, mult, p_ref, ssum_ref):
    e = _leaky(_dot(es, p_ref[0:N, :]) + _dot(ed, p_ref[N:2 * N, :]))
    c = e.max(axis=-1, keepdims=True)
    ee = mult * jnp.exp(e - c)
    den = _dot(ee, ssum_ref[...])
    r = 1.0 / (den + 1e-16)
    r100 = jax.lax.dot_general(r, ssum_ref[...], (((1,), (1,)), ((), ())),
                               preferred_element_type=jnp.float32)
    alpha = ee * r100
    alpha3 = alpha.reshape(vals3.shape[0], N, N)
    return jax.lax.dot_general(
        alpha3, vals3, (((1,), (1,)), ((0,), (0,))),
        preferred_element_type=jnp.float32)


def _body(xb_ref, nf_ref,
          p_ref, ssum_ref, o100_ref, om32_ref, om64_ref, om256_ref, on10_ref,
          we_ref, be_ref, gne_ref, bne_ref,
          wg1bd_ref, wsd1_ref, bg1_ref, g1_ref, b1_ref,
          wg2_ref, wsd2_ref, bg2_ref, g2_ref, b2_ref,
          wp_ref, bp_ref,
          wr_ref, br_ref, gr_ref, brb_ref,
          wt_ref, bt_ref, gt_ref, btb_ref,
          wf_ref, bf_ref, gf_ref, bfb_ref,
          wa1_ref, ba1_ref, wa2_ref, ba2_ref, wa3_ref, ba3_ref,
          out_ref):
    xb = xb_ref[...]
    nf = nf_ref[...]

    h2d = jax.nn.relu(_dot(nf.reshape(BB * N, 4), we_ref[...]) + be_ref[...])
    h2d = _ln(h2d, gne_ref[...], bne_ref[...], om32_ref[...])

    tflat = xb[:, 0:100]
    nz = (tflat != 0.0).astype(jnp.float32)
    nnz = _dot(nz, o100_ref[...])
    lane = jax.lax.broadcasted_iota(jnp.int32, (BB, 100), 1)
    mult = nz + jnp.where(lane == 0, 100.0 - nnz, 0.0)

    identity = _dot(h2d, wp_ref[...]) + bp_ref[...]

    h2d3 = h2d.reshape(BB, N, 32)
    sc1 = _node_scores(wsd1_ref[...], h2d3)
    aggs = []
    for hd in range(4):
        agg = _gat_attention(h2d3, sc1[hd], sc1[4 + hd],
                             mult, p_ref, ssum_ref)
        aggs.append(agg)
    agg_cat = jnp.concatenate(aggs, axis=-1).reshape(BB * N, 128)
    x1 = _dot(agg_cat, wg1bd_ref[...]) + bg1_ref[...]
    x1 = _elu(_ln(x1, g1_ref[...], b1_ref[...], om256_ref[...]))

    hh23 = _dot(x1, wg2_ref[...]).reshape(BB, N, 64)
    sc2 = _node_scores(wsd2_ref[...], x1.reshape(BB, N, 256))
    acc = _gat_attention(hh23, sc2[0], sc2[1],
                         mult, p_ref, ssum_ref)
    x2 = acc.reshape(BB * N, 64) + bg2_ref[...]
    x2 = _ln(x2, g2_ref[...], b2_ref[...], om64_ref[...])

    out = _elu(x2 + identity)
    g = out.reshape(BB, N, 64).mean(axis=1)

    r = _ln(jax.nn.relu(_dot(xb[:, 130:140], wr_ref[...]) + br_ref[...]),
            gr_ref[...], brb_ref[...], om64_ref[...])
    t = _ln(jax.nn.relu(_dot(xb[:, 240:245], wt_ref[...]) + bt_ref[...]),
            gt_ref[...], btb_ref[...], om32_ref[...])

    comb = jnp.concatenate([g, r, t], axis=1)
    feat = _ln(jax.nn.relu(_dot(comb, wf_ref[...]) + bf_ref[...]),
               gf_ref[...], bfb_ref[...], om256_ref[...])
    h1 = jax.nn.relu(_dot(feat, wa1_ref[...]) + ba1_ref[...])
    h2f = jax.nn.relu(_dot(h1, wa2_ref[...]) + ba2_ref[...])
    out_ref[...] = _dot(h2f, wa3_ref[...]) + ba3_ref[...]


def _pattern_consts():
    P = np.zeros((2 * N, N * N), np.float32)
    S = np.zeros((N * N, N), np.float32)
    for s in range(N):
        for d in range(N):
            P[s, s * N + d] = 1.0
            P[N + d, s * N + d] = 1.0
            S[s * N + d, d] = 1.0
    return jnp.asarray(P), jnp.asarray(S)


@jax.jit
def kernel(x, params):
    p = params
    B = x.shape[0]
    nf = jnp.concatenate(
        [x[:, 100:130].reshape(B, N, 3), x[:, 245:255][..., None]], axis=-1)

    P, S = _pattern_consts()
    o100 = jnp.ones((100, 1), jnp.float32)
    om32 = jnp.full((32, 1), 1.0 / 32, jnp.float32)
    om64 = jnp.full((64, 1), 1.0 / 64, jnp.float32)
    om256 = jnp.full((256, 1), 1.0 / 256, jnp.float32)
    on10 = jnp.full((1, N), 1.0 / N, jnp.float32)
    consts = [P, S, o100, om32, om64, om256, on10]

    wg1h = p['wg1'].reshape(32, 4, 64)
    ws1 = jnp.einsum('chk,hk->hc', wg1h, p['as1'])
    wd1 = jnp.einsum('chk,hk->hc', wg1h, p['ad1'])
    wsd1 = jnp.concatenate([ws1, wd1], axis=0)
    wsd2 = jnp.concatenate([p['as2'] @ p['wg2'].T,
                            p['ad2'] @ p['wg2'].T], axis=0)
    wg1bd = jnp.einsum('hg,chk->hcgk', jnp.eye(4, dtype=jnp.float32),
                       wg1h).reshape(128, 256)

    param_names = ['we', 'be', 'gne', 'bne',
                   'wg1bd', 'wsd1', 'bg1', 'g1', 'b1',
                   'wg2', 'wsd2', 'bg2', 'g2', 'b2',
                   'wp', 'bp',
                   'wr', 'br', 'gr', 'brb',
                   'wt', 'bt', 'gt', 'btb',
                   'wf', 'bf', 'gf', 'bfb',
                   'wa1', 'ba1', 'wa2', 'ba2', 'wa3', 'ba3']
    pd = dict(p, wsd1=wsd1, wsd2=wsd2, wg1bd=wg1bd)
    pvals = consts + [(pd[n].reshape(1, -1) if pd[n].ndim == 1 else pd[n])
                      for n in param_names]

    def const_spec(a):
        nd = a.ndim
        return pl.BlockSpec(a.shape, lambda i, _nd=nd: (0,) * _nd)

    in_specs = [
        pl.BlockSpec((BB, 255), lambda i: (i, 0)),
        pl.BlockSpec((BB, N, 4), lambda i: (i, 0, 0)),
    ] + [const_spec(a) for a in pvals]

    out = pl.pallas_call(
        _body,
        grid=(B // BB,),
        in_specs=in_specs,
        out_specs=pl.BlockSpec((BB, 10), lambda i: (i, 0)),
        out_shape=jax.ShapeDtypeStruct((B, 10), jnp.float32),
        compiler_params=pltpu.CompilerParams(
            dimension_semantics=("parallel",)),
    )(x, nf, *pvals)
    return out


# speedup vs baseline: 1.0060x; 1.0060x over previous
import numpy as np

import jax
import jax.numpy as jnp
from jax.experimental import pallas as pl
from jax.experimental.pallas import tpu as pltpu

N = 10
BB = 256


def _elu(x):
    return jnp.where(x > 0, x, jnp.exp(x) - 1.0)


def _leaky(x):
    return jnp.where(x > 0, x, 0.2 * x)


def _dot(a, b):
    return jax.lax.dot_general(a, b, (((1,), (0,)), ((), ())),
                               preferred_element_type=jnp.float32)


def _ln(x, g, b, om):
    m = _dot(x, om)
    d = x - m
    v = _dot(d * d, om)
    return d * jax.lax.rsqrt(v + 1e-5) * g + b


def _node_scores(wsd, feat3):
    return jax.lax.dot_general(wsd, feat3, (((1,), (2,)), ((), ())),
                               preferred_element_type=jnp.float32)


def _gat_attention(vals3, es, ed, mult, p_ref, ssum_ref):
    e = _leaky(_dot(es, p_ref[0:N, :]) + _dot(ed, p_ref[N:2 * N, :]))
    c = e.max(axis=-1, keepdims=True)
    ee = mult * jnp.exp(e - c)
    den = _dot(ee, ssum_ref[...])
    r = 1.0 / (den + 1e-16)
    r100 = jax.lax.dot_general(r, ssum_ref[...], (((1,), (1,)), ((), ())),
                               preferred_element_type=jnp.float32)
    alpha = ee * r100
    alpha3 = alpha.reshape(vals3.shape[0], N, N)
    return jax.lax.dot_general(
        alpha3, vals3, (((1,), (1,)), ((0,), (0,))),
        preferred_element_type=jnp.float32)


def _body(xb_ref, nf_ref,
          p_ref, ssum_ref, o100_ref, om32_ref, om64_ref, om256_ref,
          we_ref, be_ref, gne_ref, bne_ref,
          wg1bd_ref, wsd1_ref, bg1_ref, g1_ref, b1_ref,
          wg2_ref, wsd2_ref, bg2_ref, g2_ref, b2_ref,
          wp_ref, bp_ref,
          wr_ref, br_ref, gr_ref, brb_ref,
          wt_ref, bt_ref, gt_ref, btb_ref,
          wf_ref, bf_ref, gf_ref, bfb_ref,
          wa1_ref, ba1_ref, wa2_ref, ba2_ref, wa3_ref, ba3_ref,
          out_ref):
    xb = xb_ref[...]
    nf = nf_ref[...]

    h2d = jax.nn.relu(_dot(nf.reshape(BB * N, 4), we_ref[...]) + be_ref[...])
    h2d = _ln(h2d, gne_ref[...], bne_ref[...], om32_ref[...])

    tflat = xb[:, 0:100]
    nz = (tflat != 0.0).astype(jnp.float32)
    nnz = _dot(nz, o100_ref[...])
    lane = jax.lax.broadcasted_iota(jnp.int32, (BB, 100), 1)
    mult = nz + jnp.where(lane == 0, 100.0 - nnz, 0.0)

    identity = _dot(h2d, wp_ref[...]) + bp_ref[...]

    h2d3 = h2d.reshape(BB, N, 32)
    sc1 = _node_scores(wsd1_ref[...], h2d3)
    aggs = []
    for hd in range(4):
        agg = _gat_attention(h2d3, sc1[hd], sc1[4 + hd],
                             mult, p_ref, ssum_ref)
        aggs.append(agg)
    agg_cat = jnp.concatenate(aggs, axis=-1).reshape(BB * N, 128)
    x1 = _dot(agg_cat, wg1bd_ref[...]) + bg1_ref[...]
    x1 = _elu(_ln(x1, g1_ref[...], b1_ref[...], om256_ref[...]))

    hh23 = _dot(x1, wg2_ref[...]).reshape(BB, N, 64)
    sc2 = _node_scores(wsd2_ref[...], x1.reshape(BB, N, 256))
    acc = _gat_attention(hh23, sc2[0], sc2[1],
                         mult, p_ref, ssum_ref)
    x2 = acc.reshape(BB * N, 64) + bg2_ref[...]
    x2 = _ln(x2, g2_ref[...], b2_ref[...], om64_ref[...])

    out = _elu(x2 + identity)
    g = out.reshape(BB, N, 64).mean(axis=1)

    r = _ln(jax.nn.relu(_dot(xb[:, 130:140], wr_ref[...]) + br_ref[...]),
            gr_ref[...], brb_ref[...], om64_ref[...])
    t = _ln(jax.nn.relu(_dot(xb[:, 240:245], wt_ref[...]) + bt_ref[...]),
            gt_ref[...], btb_ref[...], om32_ref[...])

    comb = jnp.concatenate([g, r, t], axis=1)
    feat = _ln(jax.nn.relu(_dot(comb, wf_ref[...]) + bf_ref[...]),
               gf_ref[...], bfb_ref[...], om256_ref[...])
    h1 = jax.nn.relu(_dot(feat, wa1_ref[...]) + ba1_ref[...])
    h2f = jax.nn.relu(_dot(h1, wa2_ref[...]) + ba2_ref[...])
    out_ref[...] = _dot(h2f, wa3_ref[...]) + ba3_ref[...]


def _pattern_consts():
    P = np.zeros((2 * N, N * N), np.float32)
    S = np.zeros((N * N, N), np.float32)
    for s in range(N):
        for d in range(N):
            P[s, s * N + d] = 1.0
            P[N + d, s * N + d] = 1.0
            S[s * N + d, d] = 1.0
    return jnp.asarray(P), jnp.asarray(S)


@jax.jit
def kernel(x, params):
    p = params
    B = x.shape[0]
    nf = jnp.concatenate(
        [x[:, 100:130].reshape(B, N, 3), x[:, 245:255][..., None]], axis=-1)

    P, S = _pattern_consts()
    o100 = jnp.ones((100, 1), jnp.float32)
    om32 = jnp.full((32, 1), 1.0 / 32, jnp.float32)
    om64 = jnp.full((64, 1), 1.0 / 64, jnp.float32)
    om256 = jnp.full((256, 1), 1.0 / 256, jnp.float32)
    consts = [P, S, o100, om32, om64, om256]

    wg1h = p['wg1'].reshape(32, 4, 64)
    ws1 = jnp.einsum('chk,hk->hc', wg1h, p['as1'])
    wd1 = jnp.einsum('chk,hk->hc', wg1h, p['ad1'])
    wsd1 = jnp.concatenate([ws1, wd1], axis=0)
    wsd2 = jnp.concatenate([p['as2'] @ p['wg2'].T,
                            p['ad2'] @ p['wg2'].T], axis=0)
    wg1bd = jnp.einsum('hg,chk->hcgk', jnp.eye(4, dtype=jnp.float32),
                       wg1h).reshape(128, 256)

    param_names = ['we', 'be', 'gne', 'bne',
                   'wg1bd', 'wsd1', 'bg1', 'g1', 'b1',
                   'wg2', 'wsd2', 'bg2', 'g2', 'b2',
                   'wp', 'bp',
                   'wr', 'br', 'gr', 'brb',
                   'wt', 'bt', 'gt', 'btb',
                   'wf', 'bf', 'gf', 'bfb',
                   'wa1', 'ba1', 'wa2', 'ba2', 'wa3', 'ba3']
    pd = dict(p, wsd1=wsd1, wsd2=wsd2, wg1bd=wg1bd)
    pvals = consts + [(pd[n].reshape(1, -1) if pd[n].ndim == 1 else pd[n])
                      for n in param_names]

    def const_spec(a):
        nd = a.ndim
        return pl.BlockSpec(a.shape, lambda i, _nd=nd: (0,) * _nd)

    in_specs = [
        pl.BlockSpec((BB, 255), lambda i: (i, 0)),
        pl.BlockSpec((BB, N, 4), lambda i: (i, 0, 0)),
    ] + [const_spec(a) for a in pvals]

    out = pl.pallas_call(
        _body,
        grid=(B // BB,),
        in_specs=in_specs,
        out_specs=pl.BlockSpec((BB, 10), lambda i: (i, 0)),
        out_shape=jax.ShapeDtypeStruct((B, 10), jnp.float32),
        compiler_params=pltpu.CompilerParams(
            dimension_semantics=("parallel",)),
    )(x, nf, *pvals)
    return out
